# Initial kernel scaffold; baseline (speedup 1.0000x reference)
import jax
import jax.numpy as jnp
from jax.experimental import pallas as pl


def kernel(normal_features, noise, edge_index, batch, W_l, W_r, b_sage, gamma, beta, fc1_W, fc1_b, fc2_W, fc2_b, fc3_W, fc3_b, fc4_W, fc4_b):
    raise NotImplementedError("write your pallas kernel here")


# speedup vs baseline: 7.3153x; 7.3153x over previous
import functools

import jax
import jax.numpy as jnp
from jax import lax
from jax.experimental import pallas as pl
from jax.experimental.pallas import tpu as pltpu
from jax.experimental.pallas import tpu_sc as plsc

N = 10000
E = 320000
IN_DIM = 160
HALF = 80
FEAT = 128

EDGES_PER_TILE = E // 16
CHUNK = 80
NCHUNKS = EDGES_PER_TILE // CHUNK
ROWS_PER_SUB = N // 16
CNT_W = 16


def _sc_aggregate(x0, x1, src_r, dst_r):
  mesh = plsc.VectorSubcoreMesh(core_axis_name="c", subcore_axis_name="s")
  zrow = jnp.zeros((ROWS_PER_SUB, HALF), jnp.float32)
  zcnt = jnp.zeros((ROWS_PER_SUB, CNT_W), jnp.float32)
  ones = jnp.ones((CHUNK, CNT_W), jnp.float32)

  @functools.partial(
      pl.kernel,
      out_type=[
          jax.ShapeDtypeStruct((2, N, HALF), jnp.float32),
          jax.ShapeDtypeStruct((N, CNT_W), jnp.float32),
      ],
      mesh=mesh,
      compiler_params=pltpu.CompilerParams(use_tc_tiling_on_sc=False),
      scratch_types=[
          pltpu.VMEM((NCHUNKS, CHUNK), jnp.int32),
          pltpu.VMEM((NCHUNKS, CHUNK), jnp.int32),
          pltpu.VMEM((CHUNK, HALF), jnp.float32),
          pltpu.VMEM((CHUNK, HALF), jnp.float32),
          pltpu.VMEM((CHUNK, CNT_W), jnp.float32),
          pltpu.VMEM_SHARED((N, HALF), jnp.float32),
          pltpu.VMEM_SHARED((N, CNT_W), jnp.float32),
          pltpu.SemaphoreType.DMA,
          pltpu.SemaphoreType.DMA,
      ],
  )
  def agg_kernel(x0_hbm, x1_hbm, src_hbm, dst_hbm, zrow_hbm, zcnt_hbm,
                 ones_hbm, agg_out, cnt_out,
                 src_v, dst_v, rows0, rows1, ones_v, agg_s, cnt_s,
                 sem0, sem1):
    c = lax.axis_index("c")
    s = lax.axis_index("s")
    base = s * ROWS_PER_SUB

    pltpu.sync_copy(src_hbm.at[s], src_v)
    pltpu.sync_copy(dst_hbm.at[s], dst_v)
    pltpu.sync_copy(ones_hbm, ones_v)

    pltpu.sync_copy(zrow_hbm, agg_s.at[pl.ds(base, ROWS_PER_SUB)])
    pltpu.sync_copy(zcnt_hbm, cnt_s.at[pl.ds(base, ROWS_PER_SUB)])

    def start_gather(j, buf, sem):
      @pl.when(c == 0)
      def _():
        pltpu.make_async_copy(x0_hbm.at[src_v.at[j]], buf, sem).start()
      @pl.when(c == 1)
      def _():
        pltpu.make_async_copy(x1_hbm.at[src_v.at[j]], buf, sem).start()

    def wait_gather(j, buf, sem):
      pltpu.make_async_copy(x0_hbm.at[src_v.at[j]], buf, sem).wait()

    def scatter(j, buf):
      pltpu.sync_copy(buf, agg_s.at[dst_v.at[j]], add=True)
      @pl.when(c == 0)
      def _():
        pltpu.sync_copy(ones_v, cnt_s.at[dst_v.at[j]], add=True)

    start_gather(0, rows0, sem0)
    plsc.subcore_barrier()

    def body(g, carry):
      j0 = 2 * g
      wait_gather(j0, rows0, sem0)
      start_gather(j0 + 1, rows1, sem1)
      scatter(j0, rows0)
      wait_gather(j0 + 1, rows1, sem1)
      start_gather(j0 + 2, rows0, sem0)
      scatter(j0 + 1, rows1)
      return carry

    lax.fori_loop(0, NCHUNKS // 2 - 1, body, 0)
    wait_gather(NCHUNKS - 2, rows0, sem0)
    start_gather(NCHUNKS - 1, rows1, sem1)
    scatter(NCHUNKS - 2, rows0)
    wait_gather(NCHUNKS - 1, rows1, sem1)
    scatter(NCHUNKS - 1, rows1)

    plsc.subcore_barrier()
    pltpu.sync_copy(agg_s.at[pl.ds(base, ROWS_PER_SUB)],
                    agg_out.at[c, pl.ds(base, ROWS_PER_SUB)])
    @pl.when(c == 0)
    def _():
      pltpu.sync_copy(cnt_s.at[pl.ds(base, ROWS_PER_SUB)],
                      cnt_out.at[pl.ds(base, ROWS_PER_SUB)])

  return agg_kernel(x0, x1, src_r, dst_r, zrow, zcnt, ones)


def _tc_body(agg_ref, cnt_ref, x_ref, wl_ref, wr_ref, b_ref, g_ref, be_ref,
             w1_ref, b1_ref, w2_ref, b2_ref, w3_ref, b3_ref, w4_ref, b4_ref,
             out_ref):
  inv = 1.0 / jnp.maximum(cnt_ref[:, 0:1], 1.0)
  h = (jnp.dot(agg_ref[0] * inv, wl_ref[0:HALF],
               preferred_element_type=jnp.float32)
       + jnp.dot(agg_ref[1] * inv, wl_ref[HALF:IN_DIM],
                 preferred_element_type=jnp.float32)
       + jnp.dot(x_ref[...], wr_ref[...], preferred_element_type=jnp.float32)
       + b_ref[...])
  mu = jnp.mean(h, axis=0, keepdims=True)
  var = jnp.mean(h * h, axis=0, keepdims=True) - mu * mu
  h = (h - mu) * (g_ref[...] * lax.rsqrt(var + 1e-5)) + be_ref[...]
  h = jnp.where(h > 0, h, 0.2 * h)
  h = jnp.dot(h, w1_ref[...], preferred_element_type=jnp.float32) + b1_ref[...]
  h = jnp.where(h > 0, h, 0.2 * h)
  h = jnp.dot(h, w2_ref[...], preferred_element_type=jnp.float32) + b2_ref[...]
  h = jnp.where(h > 0, h, 0.2 * h)
  h = jnp.dot(h, w3_ref[...], preferred_element_type=jnp.float32) + b3_ref[...]
  h = jnp.where(h > 0, h, 0.2 * h)
  h = jnp.dot(h, w4_ref[...], preferred_element_type=jnp.float32) + b4_ref[...]
  out_ref[...] = jnp.where(h > 0, h, 0.2 * h)


def kernel(normal_features, noise, edge_index, batch, W_l, W_r, b_sage,
           gamma, beta, fc1_W, fc1_b, fc2_W, fc2_b, fc3_W, fc3_b,
           fc4_W, fc4_b):
  del batch
  x = jnp.concatenate([normal_features, noise], axis=1)
  x0 = x[:, :HALF]
  x1 = x[:, HALF:]
  src_r = edge_index[0].reshape(16, NCHUNKS, CHUNK)
  dst_r = edge_index[1].reshape(16, NCHUNKS, CHUNK)
  agg, cnt = _sc_aggregate(x0, x1, src_r, dst_r)
  return pl.pallas_call(
      _tc_body,
      out_shape=jax.ShapeDtypeStruct((N, FEAT), jnp.float32),
  )(agg, cnt, x, W_l, W_r, b_sage, gamma, beta,
    fc1_W, fc1_b, fc2_W, fc2_b, fc3_W, fc3_b, fc4_W, fc4_b)
